# baseline (speedup 1.0000x reference)
import functools

import jax
import jax.numpy as jnp
from jax import lax
from jax.experimental import pallas as pl
from jax.experimental.pallas import tpu as pltpu
from jax.experimental.pallas import tpu_sc as plsc

_NUM_EXPERTS = 16
_TOP_K = 2
_LANES = 128
_NEG = -1e30

_NC = 2
_NS = 16
_NW = _NC * _NS
_VL = 16

_TILE = 2048


def _logits_body(tok_per, x_ref, w_ref, b_ref, lt_ref):
    lt = jax.lax.dot_general(
        w_ref[...], x_ref[...], (((0,), (1,)), ((), ())),
        preferred_element_type=jnp.float32)[:_NUM_EXPERTS, :] + b_ref[:, 0:1]
    slabs = lt_ref.shape[0]
    for s in range(slabs):
        lt_ref[s] = lt[:, s * tok_per:(s + 1) * tok_per]


def _logits_tc(xf, Wp, bcol, tok_per):
    T, D = xf.shape
    num_tiles = T // _TILE
    slabs = _TILE // tok_per
    return pl.pallas_call(
        functools.partial(_logits_body, tok_per),
        grid=(num_tiles,),
        in_specs=[
            pl.BlockSpec((_TILE, D), lambda i: (i, 0)),
            pl.BlockSpec((D, _LANES), lambda i: (0, 0)),
            pl.BlockSpec((_NUM_EXPERTS, 8), lambda i: (0, 0)),
        ],
        out_specs=pl.BlockSpec((slabs, _NUM_EXPERTS, tok_per),
                               lambda i: (i, 0, 0)),
        out_shape=jax.ShapeDtypeStruct((T // tok_per, _NUM_EXPERTS, tok_per),
                                       jnp.float32),
    )(xf, Wp, bcol)


def _route_body(tok_per, lt_hbm, i1_hbm, i2_hbm, w1_hbm, w2_hbm, cnt_hbm,
                lt_v, i1_v, i2_v, w1_v, w2_v, cnt_v, sem):
    wid = lax.axis_index("s") * _NC + lax.axis_index("c")
    base = wid * tok_per
    pltpu.sync_copy(lt_hbm.at[wid], lt_v)

    for e in range(_NUM_EXPERTS):
        cnt_v[e] = jnp.zeros((_VL,), jnp.float32)

    def chunk(j, carry):
        m1 = jnp.full((_VL,), _NEG, jnp.float32)
        m2 = jnp.full((_VL,), _NEG, jnp.float32)
        i1 = jnp.zeros((_VL,), jnp.int32)
        i2 = jnp.zeros((_VL,), jnp.int32)
        for e in range(_NUM_EXPERTS):
            v = lt_v[e, pl.ds(j * _VL, _VL)]
            ev = jnp.full((_VL,), e, jnp.int32)
            gt1 = v > m1
            gt2 = v > m2
            m2n = jnp.where(gt1, m1, jnp.where(gt2, v, m2))
            i2n = jnp.where(gt1, i1, jnp.where(gt2, ev, i2))
            m1 = jnp.where(gt1, v, m1)
            i1 = jnp.where(gt1, ev, i1)
            m2, i2 = m2n, i2n
        w1 = 1.0 / (1.0 + jnp.exp(m2 - m1))
        sl = pl.ds(j * _VL, _VL)
        i1_v[sl] = i1
        i2_v[sl] = i2
        w1_v[sl] = w1
        w2_v[sl] = 1.0 - w1
        for e in range(_NUM_EXPERTS):
            ev = jnp.full((_VL,), e, jnp.int32)
            hits = (jnp.where(i1 == ev, 1.0, 0.0) +
                    jnp.where(i2 == ev, 1.0, 0.0))
            cnt_v[e] = cnt_v[e] + hits
        return carry

    lax.fori_loop(0, tok_per // _VL, chunk, 0)

    row = pl.ds(base, tok_per)
    cps = [
        pltpu.async_copy(i1_v, i1_hbm.at[row], sem),
        pltpu.async_copy(i2_v, i2_hbm.at[row], sem),
        pltpu.async_copy(w1_v, w1_hbm.at[row], sem),
        pltpu.async_copy(w2_v, w2_hbm.at[row], sem),
        pltpu.async_copy(cnt_v, cnt_hbm.at[wid], sem),
    ]
    for cp in cps:
        cp.wait()


def _route_sc(lt, T):
    tok_per = T // _NW
    mesh = plsc.VectorSubcoreMesh(core_axis_name="c", subcore_axis_name="s",
                                  num_cores=_NC, num_subcores=_NS)
    return pl.kernel(
        functools.partial(_route_body, tok_per),
        out_type=[
            jax.ShapeDtypeStruct((T,), jnp.int32),
            jax.ShapeDtypeStruct((T,), jnp.int32),
            jax.ShapeDtypeStruct((T,), jnp.float32),
            jax.ShapeDtypeStruct((T,), jnp.float32),
            jax.ShapeDtypeStruct((_NW, _NUM_EXPERTS, _VL), jnp.float32),
        ],
        mesh=mesh,
        scratch_types=[
            pltpu.VMEM((_NUM_EXPERTS, tok_per), jnp.float32),
            pltpu.VMEM((tok_per,), jnp.int32),
            pltpu.VMEM((tok_per,), jnp.int32),
            pltpu.VMEM((tok_per,), jnp.float32),
            pltpu.VMEM((tok_per,), jnp.float32),
            pltpu.VMEM((_NUM_EXPERTS, _VL), jnp.float32),
            pltpu.SemaphoreType.DMA,
        ],
    )(lt)


@jax.jit
def kernel(x, W, b):
    B, S, D = x.shape
    T = B * S
    xf = x.reshape(T, D)

    Wp = jnp.zeros((D, _LANES), jnp.float32).at[:, :_NUM_EXPERTS].set(W)
    bcol = jnp.zeros((_NUM_EXPERTS, 8), jnp.float32).at[:, 0].set(b)

    lt = _logits_tc(xf, Wp, bcol, T // _NW)
    i1, i2, w1, w2, cnt = _route_sc(lt, T)

    usage = jnp.sum(cnt, axis=(0, 2))
    mean = jnp.float32(_TOP_K * T / _NUM_EXPERTS)
    var = jnp.sum((usage - mean) ** 2) / (_NUM_EXPERTS - 1)
    loss = jnp.sqrt(var) / (mean + 1e-10) * 0.01

    idx = jnp.stack([i1, i2], axis=-1).reshape(B, S, _TOP_K)
    wgt = jnp.stack([w1, w2], axis=-1).reshape(B, S, _TOP_K)
    return (idx, wgt, loss)


# speedup vs baseline: 1.0569x; 1.0569x over previous
import functools

import jax
import jax.numpy as jnp
from jax import lax
from jax.experimental import pallas as pl
from jax.experimental.pallas import tpu as pltpu
from jax.experimental.pallas import tpu_sc as plsc

_NUM_EXPERTS = 16
_TOP_K = 2
_LANES = 128
_NEG = -1e30

_NC = 2
_NS = 16
_NW = _NC * _NS
_VL = 16

_TILE = 2048


def _logits_body(tok_per, x_ref, w_ref, b_ref, lt_ref):
    lt = jax.lax.dot_general(
        w_ref[...], x_ref[...], (((1,), (1,)), ((), ())),
        preferred_element_type=jnp.float32) + b_ref[:, 0:1]
    slabs = lt_ref.shape[0]
    for s in range(slabs):
        lt_ref[s] = lt[:, s * tok_per:(s + 1) * tok_per]


def _logits_tc(xf, Wp, bcol, tok_per):
    T, D = xf.shape
    num_tiles = T // _TILE
    slabs = _TILE // tok_per
    return pl.pallas_call(
        functools.partial(_logits_body, tok_per),
        grid=(num_tiles,),
        in_specs=[
            pl.BlockSpec((_TILE, D), lambda i: (i, 0)),
            pl.BlockSpec((_NUM_EXPERTS, D), lambda i: (0, 0)),
            pl.BlockSpec((_NUM_EXPERTS, 8), lambda i: (0, 0)),
        ],
        out_specs=pl.BlockSpec((slabs, _NUM_EXPERTS, tok_per),
                               lambda i: (i, 0, 0)),
        out_shape=jax.ShapeDtypeStruct((T // tok_per, _NUM_EXPERTS, tok_per),
                                       jnp.float32),
    )(xf, Wp, bcol)


def _route_body(tok_per, lt_hbm, i1_hbm, i2_hbm, w1_hbm, w2_hbm, cnt_hbm,
                lt_v, i1_v, i2_v, w1_v, w2_v, cnt_v, sem):
    wid = lax.axis_index("s") * _NC + lax.axis_index("c")
    base = wid * tok_per
    pltpu.sync_copy(lt_hbm.at[wid], lt_v)

    for e in range(_NUM_EXPERTS):
        cnt_v[e] = jnp.zeros((_VL,), jnp.float32)

    def chunk(j, carry):
        m1 = jnp.full((_VL,), _NEG, jnp.float32)
        m2 = jnp.full((_VL,), _NEG, jnp.float32)
        i1 = jnp.zeros((_VL,), jnp.int32)
        i2 = jnp.zeros((_VL,), jnp.int32)
        for e in range(_NUM_EXPERTS):
            v = lt_v[e, pl.ds(j * _VL, _VL)]
            ev = jnp.full((_VL,), e, jnp.int32)
            gt1 = v > m1
            gt2 = v > m2
            m2n = jnp.where(gt1, m1, jnp.where(gt2, v, m2))
            i2n = jnp.where(gt1, i1, jnp.where(gt2, ev, i2))
            m1 = jnp.where(gt1, v, m1)
            i1 = jnp.where(gt1, ev, i1)
            m2, i2 = m2n, i2n
        w1 = 1.0 / (1.0 + jnp.exp(m2 - m1))
        sl = pl.ds(j * _VL, _VL)
        i1_v[sl] = i1
        i2_v[sl] = i2
        w1_v[sl] = w1
        w2_v[sl] = 1.0 - w1
        for e in range(_NUM_EXPERTS):
            ev = jnp.full((_VL,), e, jnp.int32)
            hits = (jnp.where(i1 == ev, 1.0, 0.0) +
                    jnp.where(i2 == ev, 1.0, 0.0))
            cnt_v[e] = cnt_v[e] + hits
        return carry

    lax.fori_loop(0, tok_per // _VL, chunk, 0)

    row = pl.ds(base, tok_per)
    cps = [
        pltpu.async_copy(i1_v, i1_hbm.at[row], sem),
        pltpu.async_copy(i2_v, i2_hbm.at[row], sem),
        pltpu.async_copy(w1_v, w1_hbm.at[row], sem),
        pltpu.async_copy(w2_v, w2_hbm.at[row], sem),
        pltpu.async_copy(cnt_v, cnt_hbm.at[wid], sem),
    ]
    for cp in cps:
        cp.wait()


def _route_sc(lt, T):
    tok_per = T // _NW
    mesh = plsc.VectorSubcoreMesh(core_axis_name="c", subcore_axis_name="s",
                                  num_cores=_NC, num_subcores=_NS)
    return pl.kernel(
        functools.partial(_route_body, tok_per),
        out_type=[
            jax.ShapeDtypeStruct((T,), jnp.int32),
            jax.ShapeDtypeStruct((T,), jnp.int32),
            jax.ShapeDtypeStruct((T,), jnp.float32),
            jax.ShapeDtypeStruct((T,), jnp.float32),
            jax.ShapeDtypeStruct((_NW, _NUM_EXPERTS, _VL), jnp.float32),
        ],
        mesh=mesh,
        scratch_types=[
            pltpu.VMEM((_NUM_EXPERTS, tok_per), jnp.float32),
            pltpu.VMEM((tok_per,), jnp.int32),
            pltpu.VMEM((tok_per,), jnp.int32),
            pltpu.VMEM((tok_per,), jnp.float32),
            pltpu.VMEM((tok_per,), jnp.float32),
            pltpu.VMEM((_NUM_EXPERTS, _VL), jnp.float32),
            pltpu.SemaphoreType.DMA,
        ],
    )(lt)


@jax.jit
def kernel(x, W, b):
    B, S, D = x.shape
    T = B * S
    xf = x.reshape(T, D)

    Wp = W.T
    bcol = jnp.zeros((_NUM_EXPERTS, 8), jnp.float32).at[:, 0].set(b)

    lt = _logits_tc(xf, Wp, bcol, T // _NW)
    i1, i2, w1, w2, cnt = _route_sc(lt, T)

    usage = jnp.sum(cnt, axis=(0, 2))
    mean = jnp.float32(_TOP_K * T / _NUM_EXPERTS)
    var = jnp.sum((usage - mean) ** 2) / (_NUM_EXPERTS - 1)
    loss = jnp.sqrt(var) / (mean + 1e-10) * 0.01

    idx = jnp.stack([i1, i2], axis=-1).reshape(B, S, _TOP_K)
    wgt = jnp.stack([w1, w2], axis=-1).reshape(B, S, _TOP_K)
    return (idx, wgt, loss)
